# baseline (speedup 1.0000x reference)
import functools

import jax
import jax.numpy as jnp
from jax import lax
from jax.experimental import pallas as pl
from jax.experimental.pallas import tpu as pltpu
from jax.experimental.pallas import tpu_sc as plsc

_N = 10000
_E = 320000
_H = 128
_BN = 1000
_BE = 2000
_PARTS = 5

_INTERPRET = False


def _ln(h, g, b):
    m = jnp.mean(h, axis=-1, keepdims=True)
    v = jnp.mean((h - m) ** 2, axis=-1, keepdims=True)
    return (h - m) * jax.lax.rsqrt(v + 1e-5) * g + b


def _full(x):
    return pl.BlockSpec(x.shape, lambda i: tuple(0 for _ in x.shape))


def _enc_nodes_body(wp, mp, phi, sphi, nt, mat, wu, wphi, wsphi, wnt, wmat,
                    b1, w2, b2, g, bta, wa, wb, xh_o, a_o, b_o):
    u = wp[...] - mp[...]
    h = (jnp.dot(u, wu[...]) + phi[...] * wphi[...] + sphi[...] * wsphi[...]
         + jnp.dot(nt[...], wnt[...]) + jnp.dot(mat[...], wmat[...]) + b1[...])
    h = jax.nn.relu(h)
    h2 = jnp.dot(h, w2[...]) + b2[...]
    xh = _ln(h2, g[...], bta[...])
    xh_o[...] = xh
    a_o[...] = jnp.dot(xh, wa[...])
    b_o[...] = jnp.dot(xh, wb[...])


def _encode_nodes(wp, mp, phi, sphi, nt, mat, wu, wphi, wsphi, wnt, wmat,
                  b1, w2, b2, g, bta, wa, wb):
    grid = (_N // _BN,)
    blk = lambda d: pl.BlockSpec((_BN, d), lambda i: (i, 0))
    ins = [wp, mp, phi, sphi, nt, mat, wu, wphi, wsphi, wnt, wmat,
           b1, w2, b2, g, bta, wa, wb]
    in_specs = [blk(3), blk(3), blk(1), blk(1), blk(9)] + [_full(x) for x in ins[5:]]
    out_sd = jax.ShapeDtypeStruct((_N, _H), jnp.float32)
    return pl.pallas_call(
        _enc_nodes_body, grid=grid, in_specs=in_specs,
        out_specs=[blk(_H)] * 3, out_shape=[out_sd] * 3,
        interpret=_INTERPRET,
    )(*ins)


def _enc_edges_body(d, w1p, wdist, wdistw, b1, w2, b2, g, bta, eh_o):
    dd = d[...]
    sq = dd * dd
    lane = lax.broadcasted_iota(jnp.int32, (1, _H), 1)
    m0 = (lane < 3).astype(jnp.float32)
    m1 = ((lane >= 3) & (lane < 6)).astype(jnp.float32)
    dist = jnp.sqrt(jnp.sum(sq * m0, axis=1, keepdims=True))
    distw = jnp.sqrt(jnp.sum(sq * m1, axis=1, keepdims=True))
    h = jnp.dot(dd, w1p[...]) + dist * wdist[...] + distw * wdistw[...] + b1[...]
    h = jax.nn.relu(h)
    h2 = jnp.dot(h, w2[...]) + b2[...]
    eh_o[...] = _ln(h2, g[...], bta[...]).astype(jnp.bfloat16)


def _encode_edges(d, w1p, wdist, wdistw, b1, w2, b2, g, bta):
    ep = d.shape[0]
    grid = (ep // _BE,)
    blk = lambda w: pl.BlockSpec((_BE, w), lambda i: (i, 0))
    ins = [d, w1p, wdist, wdistw, b1, w2, b2, g, bta]
    in_specs = [blk(_H)] + [_full(x) for x in ins[1:]]
    return pl.pallas_call(
        _enc_edges_body, grid=grid, in_specs=in_specs,
        out_specs=blk(_H),
        out_shape=jax.ShapeDtypeStruct((ep, _H), jnp.bfloat16),
        interpret=_INTERPRET,
    )(*ins)


def _edge_mlp_body(gsum, eh, w1c, b1, w2, b2, g, bta, ne_o, eo_o):
    ehv = eh[...].astype(jnp.float32)
    h = jax.nn.relu(gsum[...] + jnp.dot(ehv, w1c[...]) + b1[...])
    h2 = jax.nn.relu(jnp.dot(h, w2[...]) + b2[...])
    ne = _ln(h2, g[...], bta[...])
    ne_o[...] = ne
    if eo_o is not None:
        eo_o[...] = (ne + ehv).astype(jnp.bfloat16)


def _edge_mlp(gsum, eh, w1c, b1, w2, b2, g, bta, want_eh_out):
    ep = gsum.shape[0]
    grid = (ep // _BE,)
    blk = pl.BlockSpec((_BE, _H), lambda i: (i, 0))
    ins = [gsum, eh, w1c, b1, w2, b2, g, bta]
    in_specs = [blk, blk] + [_full(x) for x in ins[2:]]
    out_sd = jax.ShapeDtypeStruct((ep, _H), jnp.float32)
    out_sd_bf = jax.ShapeDtypeStruct((ep, _H), jnp.bfloat16)
    if want_eh_out:
        body = _edge_mlp_body
        out_specs, out_shape = [blk, blk], [out_sd, out_sd_bf]
    else:
        body = functools.partial(_edge_mlp_body, eo_o=None)
        out_specs, out_shape = blk, out_sd
    return pl.pallas_call(
        body, grid=grid, in_specs=in_specs,
        out_specs=out_specs, out_shape=out_shape,
        interpret=_INTERPRET,
    )(*ins)


def _sum_refs(refs):
    acc = refs[0][...]
    for r in refs[1:]:
        acc = acc + r[...]
    return acc


def _node_mlp(xh, ps, w1a, w1b, b1, w2, b2, g, bta, wa, wb):
    np_ = len(ps)

    def body(*refs):
        xh_r = refs[0]
        p_refs = refs[1:1 + np_]
        (w1a_r, w1b_r, b1_r, w2_r, b2_r, g_r, bta_r, wa_r, wb_r,
         nx_o, a_o, b_o) = refs[1 + np_:]
        xhv = xh_r[...]
        aggr = _sum_refs(p_refs)
        h = jax.nn.relu(jnp.dot(xhv, w1a_r[...]) + jnp.dot(aggr, w1b_r[...])
                        + b1_r[...])
        h2 = jax.nn.relu(jnp.dot(h, w2_r[...]) + b2_r[...])
        nx = _ln(h2, g_r[...], bta_r[...]) + xhv
        nx_o[...] = nx
        a_o[...] = jnp.dot(nx, wa_r[...])
        b_o[...] = jnp.dot(nx, wb_r[...])

    grid = (_N // _BN,)
    blk = pl.BlockSpec((_BN, _H), lambda i: (i, 0))
    ins = [xh] + list(ps) + [w1a, w1b, b1, w2, b2, g, bta, wa, wb]
    in_specs = [blk] * (1 + np_) + [_full(x) for x in ins[1 + np_:]]
    out_sd = jax.ShapeDtypeStruct((_N, _H), jnp.float32)
    return pl.pallas_call(
        body, grid=grid, in_specs=in_specs,
        out_specs=[blk] * 3, out_shape=[out_sd] * 3,
        interpret=_INTERPRET,
    )(*ins)


def _node_dec(xh, ps, w1a, w1b, b1, w2, b2, g, bta, wd1, bd1, wd2, bd2):
    np_ = len(ps)

    def body(*refs):
        xh_r = refs[0]
        p_refs = refs[1:1 + np_]
        (w1a_r, w1b_r, b1_r, w2_r, b2_r, g_r, bta_r,
         wd1_r, bd1_r, wd2_r, bd2_r, out_o) = refs[1 + np_:]
        xhv = xh_r[...]
        aggr = _sum_refs(p_refs)
        h = jax.nn.relu(jnp.dot(xhv, w1a_r[...]) + jnp.dot(aggr, w1b_r[...])
                        + b1_r[...])
        h2 = jax.nn.relu(jnp.dot(h, w2_r[...]) + b2_r[...])
        nx = _ln(h2, g_r[...], bta_r[...]) + xhv
        dh = jax.nn.relu(jnp.dot(nx, wd1_r[...]) + bd1_r[...])
        out_o[...] = jnp.dot(dh, wd2_r[...]) + bd2_r[...]

    grid = (_N // _BN,)
    blk = pl.BlockSpec((_BN, _H), lambda i: (i, 0))
    ins = [xh] + list(ps) + [w1a, w1b, b1, w2, b2, g, bta, wd1, bd1, wd2, bd2]
    in_specs = [blk] * (1 + np_) + [_full(x) for x in ins[1 + np_:]]
    return pl.pallas_call(
        body, grid=grid, in_specs=in_specs,
        out_specs=pl.BlockSpec((_BN, 3), lambda i: (i, 0)),
        out_shape=jax.ShapeDtypeStruct((_N, 3), jnp.float32),
        interpret=_INTERPRET,
    )(*ins)


_NC = 2
_NS = 16
_NW = _NC * _NS


@functools.lru_cache(maxsize=None)
def _make_gather_combine(subtract, ch, nchunk):
    mesh = plsc.VectorSubcoreMesh(core_axis_name="c", subcore_axis_name="s")
    epw = ch * nchunk
    e_part = epw * _NW

    def body(ta, tb, s3, r3, out, sbuf, rbuf, ra0, rb0, ra1, rb1,
             sa0, sb0, sa1, sb1):
        wid = lax.axis_index("s") * _NC + lax.axis_index("c")
        base0 = wid * epw
        pltpu.sync_copy(s3.at[wid], sbuf)
        pltpu.sync_copy(r3.at[wid], rbuf)

        ras = (ra0, ra1)
        rbs = (rb0, rb1)
        sas = (sa0, sa1)
        sbs = (sb0, sb1)

        def issue(c, p):
            pltpu.async_copy(ta.at[sbuf.at[c]], ras[p], sas[p])
            pltpu.async_copy(tb.at[rbuf.at[c]], rbs[p], sbs[p])

        def process(c, p):
            pltpu.make_async_copy(ta.at[sbuf.at[c]], ras[p], sas[p]).wait()
            pltpu.make_async_copy(tb.at[rbuf.at[c]], rbs[p], sbs[p]).wait()
            ra, rb = ras[p], rbs[p]

            def row(i, carry2):
                for k in range(4):
                    for j in range(_H // 16):
                        sl = pl.ds(j * 16, 16)
                        if subtract:
                            ra[i * 4 + k, sl] = ra[i * 4 + k, sl] - rb[i * 4 + k, sl]
                        else:
                            ra[i * 4 + k, sl] = ra[i * 4 + k, sl] + rb[i * 4 + k, sl]
                return carry2

            lax.fori_loop(0, ch // 4, row, 0)
            pltpu.sync_copy(ra, out.at[pl.ds(base0 + c * ch, ch)])

            @pl.when(c + 2 < nchunk)
            def _():
                issue(c + 2, p)

        issue(0, 0)
        issue(1, 1)

        def pair(i, carry):
            process(i * 2, 0)
            process(i * 2 + 1, 1)
            return carry

        lax.fori_loop(0, nchunk // 2, pair, 0)
        if nchunk % 2:
            process(nchunk - 1, 0)

    return pl.kernel(
        body,
        out_type=jax.ShapeDtypeStruct((e_part, _H), jnp.float32),
        mesh=mesh,
        scratch_types=[
            pltpu.VMEM((nchunk, ch), jnp.int32),
            pltpu.VMEM((nchunk, ch), jnp.int32),
            pltpu.VMEM((ch, _H), jnp.float32),
            pltpu.VMEM((ch, _H), jnp.float32),
            pltpu.VMEM((ch, _H), jnp.float32),
            pltpu.VMEM((ch, _H), jnp.float32),
            pltpu.SemaphoreType.DMA,
            pltpu.SemaphoreType.DMA,
            pltpu.SemaphoreType.DMA,
            pltpu.SemaphoreType.DMA,
        ],
    )


def _gather_combine(ta, tb, s3, r3, subtract):
    nchunk, ch = s3.shape[1], s3.shape[2]
    return _make_gather_combine(subtract, ch, nchunk)(ta, tb, s3, r3)


_DRT = 10
_DRW = _N // _DRT
_ZR = 40


@functools.lru_cache(maxsize=None)
def _make_scatter_add(ch, nchunk):
    mesh = plsc.VectorSubcoreMesh(core_axis_name="c", subcore_axis_name="s")
    epw = ch * nchunk

    def body(rows_hbm, r3, out, acc, ribuf, row_0, row_1, zbuf, sl0, sl1):
        cid = lax.axis_index("c")
        sid = lax.axis_index("s")
        wid = sid * _NC + cid
        base0 = wid * epw
        pltpu.sync_copy(r3.at[wid], ribuf)
        zv = jnp.zeros((16,), jnp.float32)

        def zrow(i, carry):
            for j in range(_H // 16):
                zbuf[i, pl.ds(j * 16, 16)] = zv
            return carry

        lax.fori_loop(0, _ZR, zrow, 0)
        row0 = sid * _DRW

        @pl.when(sid < _DRT)
        def _zero():
            for k in range(_DRW // _ZR):
                pltpu.sync_copy(zbuf, acc.at[pl.ds(row0 + k * _ZR, _ZR)])

        plsc.subcore_barrier()

        rows = (row_0, row_1)
        sems = (sl0, sl1)

        def issue(c, p):
            pltpu.async_copy(rows_hbm.at[pl.ds(base0 + c * ch, ch)],
                             rows[p], sems[p])

        def process(c, p):
            pltpu.make_async_copy(rows_hbm.at[pl.ds(base0 + c * ch, ch)],
                                  rows[p], sems[p]).wait()
            pltpu.sync_copy(rows[p], acc.at[ribuf.at[c]], add=True)

            @pl.when(c + 2 < nchunk)
            def _():
                issue(c + 2, p)

        issue(0, 0)
        issue(1, 1)

        def pair(i, carry):
            process(i * 2, 0)
            process(i * 2 + 1, 1)
            return carry

        lax.fori_loop(0, nchunk // 2, pair, 0)
        if nchunk % 2:
            process(nchunk - 1, 0)
        plsc.subcore_barrier()

        @pl.when(sid < _DRT)
        def _drain():
            for k in range(_DRW // _ZR):
                sl = pl.ds(row0 + k * _ZR, _ZR)
                pltpu.sync_copy(acc.at[sl], zbuf)
                pltpu.sync_copy(zbuf, out.at[pl.ds(cid * _N + row0 + k * _ZR, _ZR)])

    return pl.kernel(
        body,
        out_type=jax.ShapeDtypeStruct((2 * _N, _H), jnp.float32),
        mesh=mesh,
        scratch_types=[
            pltpu.VMEM_SHARED((_N, _H), jnp.float32),
            pltpu.VMEM((nchunk, ch), jnp.int32),
            pltpu.VMEM((ch, _H), jnp.float32),
            pltpu.VMEM((ch, _H), jnp.float32),
            pltpu.VMEM((_ZR, _H), jnp.float32),
            pltpu.SemaphoreType.DMA,
            pltpu.SemaphoreType.DMA,
        ],
    )


def _scatter_add(rows, r3):
    nchunk, ch = r3.shape[1], r3.shape[2]
    p = _make_scatter_add(ch, nchunk)(rows, r3)
    return p[:_N], p[_N:]


def kernel(world_pos, mesh_pos, phi, swelling_phi, node_type, mat_param,
           edge_index, params):
    f32 = jnp.float32
    ep = _E // _PARTS
    chp = (ep // _NW) // 125
    nchp = ep // _NW // chp
    s_parts = [edge_index[0, k * ep:(k + 1) * ep].reshape(_NW, nchp, chp)
               for k in range(_PARTS)]
    r_parts = [edge_index[1, k * ep:(k + 1) * ep].reshape(_NW, nchp, chp)
               for k in range(_PARTS)]

    ne = params["node_enc"]
    wn1 = ne["l1"]["w"]
    wu, wphi, wsphi, wnt, wmat = (wn1[0:3], wn1[3:4], wn1[4:5], wn1[5:14],
                                  wn1[14:19])
    ee = params["edge_enc"]
    we1 = ee["l1"]["w"]
    w1p = jnp.concatenate([we1[0:3], we1[4:7], we1[8:9],
                           jnp.zeros((_H - 7, _H), f32)], axis=0)
    wdist, wdistw = we1[3:4], we1[7:8]

    procs = params["procs"]
    ew = [p["edge"] for p in procs]
    nw = [p["node"] for p in procs]
    e_w1a = [w["l1"]["w"][0:_H] for w in ew]
    e_w1b = [w["l1"]["w"][_H:2 * _H] for w in ew]
    e_w1c = [w["l1"]["w"][2 * _H:] for w in ew]
    n_w1a = [w["l1"]["w"][0:_H] for w in nw]
    n_w1b = [w["l1"]["w"][_H:] for w in nw]

    def row(x):
        return x.reshape(1, -1)

    t_tab = jnp.concatenate([mesh_pos, world_pos, phi,
                             jnp.zeros((_N, _H - 7), f32)], axis=1)
    d_parts = [_gather_combine(t_tab, t_tab, s_parts[k], r_parts[k], True)
               for k in range(_PARTS)]

    x_h, a_t, b_t = _encode_nodes(
        world_pos, mesh_pos, phi, swelling_phi, node_type,
        mat_param.reshape(1, 5), wu, wphi, wsphi, wnt, wmat,
        row(ne["l1"]["b"]), ne["l2"]["w"], row(ne["l2"]["b"]),
        row(ne["g"]), row(ne["bta"]), e_w1a[0], e_w1b[0])

    eh_parts = [_encode_edges(d, w1p, wdist, wdistw, row(ee["l1"]["b"]),
                              ee["l2"]["w"], row(ee["l2"]["b"]),
                              row(ee["g"]), row(ee["bta"]))
                for d in d_parts]

    for t in range(3):
        last = t == 2
        g_parts = [_gather_combine(a_t, b_t, s_parts[k], r_parts[k], False)
                   for k in range(_PARTS)]
        ne_parts = []
        new_eh = []
        for k in range(_PARTS):
            if last:
                nek = _edge_mlp(g_parts[k], eh_parts[k], e_w1c[t],
                                row(ew[t]["l1"]["b"]), ew[t]["l2"]["w"],
                                row(ew[t]["l2"]["b"]), row(ew[t]["g"]),
                                row(ew[t]["bta"]), False)
            else:
                nek, ehk = _edge_mlp(g_parts[k], eh_parts[k], e_w1c[t],
                                     row(ew[t]["l1"]["b"]), ew[t]["l2"]["w"],
                                     row(ew[t]["l2"]["b"]), row(ew[t]["g"]),
                                     row(ew[t]["bta"]), True)
                new_eh.append(ehk)
            ne_parts.append(nek)
        eh_parts = new_eh
        ps = []
        for k in range(_PARTS):
            p0, p1 = _scatter_add(ne_parts[k], r_parts[k])
            ps += [p0, p1]
        w = nw[t]
        if last:
            dec = params["dec"]
            out = _node_dec(x_h, ps, n_w1a[t], n_w1b[t],
                            row(w["l1"]["b"]), w["l2"]["w"], row(w["l2"]["b"]),
                            row(w["g"]), row(w["bta"]),
                            dec["l1"]["w"], row(dec["l1"]["b"]),
                            dec["l2"]["w"], row(dec["l2"]["b"]))
        else:
            x_h, a_t, b_t = _node_mlp(
                x_h, ps, n_w1a[t], n_w1b[t],
                row(w["l1"]["b"]), w["l2"]["w"], row(w["l2"]["b"]),
                row(w["g"]), row(w["bta"]), e_w1a[t + 1], e_w1b[t + 1])

    return out[None, :, :]


# speedup vs baseline: 1.5248x; 1.5248x over previous
import functools

import jax
import jax.numpy as jnp
from jax import lax
from jax.experimental import pallas as pl
from jax.experimental.pallas import tpu as pltpu
from jax.experimental.pallas import tpu_sc as plsc

_N = 10000
_E = 320000
_H = 128
_BN = 1000
_BE = 2000
_PARTS = 5

_INTERPRET = False


def _ln(h, g, b):
    m = jnp.mean(h, axis=-1, keepdims=True)
    v = jnp.mean((h - m) ** 2, axis=-1, keepdims=True)
    return (h - m) * jax.lax.rsqrt(v + 1e-5) * g + b


def _full(x):
    return pl.BlockSpec(x.shape, lambda i: tuple(0 for _ in x.shape))


def _enc_nodes_body(wp, mp, phi, sphi, nt, mat, wu, wphi, wsphi, wnt, wmat,
                    b1, w2, b2, g, bta, wa, wb, xh_o, a_o, b_o):
    u = wp[...] - mp[...]
    h = (jnp.dot(u, wu[...]) + phi[...] * wphi[...] + sphi[...] * wsphi[...]
         + jnp.dot(nt[...], wnt[...]) + jnp.dot(mat[...], wmat[...]) + b1[...])
    h = jax.nn.relu(h)
    h2 = jnp.dot(h, w2[...]) + b2[...]
    xh = _ln(h2, g[...], bta[...])
    xh_o[...] = xh
    a_o[...] = jnp.dot(xh, wa[...])
    b_o[...] = jnp.dot(xh, wb[...])


def _encode_nodes(wp, mp, phi, sphi, nt, mat, wu, wphi, wsphi, wnt, wmat,
                  b1, w2, b2, g, bta, wa, wb):
    grid = (_N // _BN,)
    blk = lambda d: pl.BlockSpec((_BN, d), lambda i: (i, 0))
    ins = [wp, mp, phi, sphi, nt, mat, wu, wphi, wsphi, wnt, wmat,
           b1, w2, b2, g, bta, wa, wb]
    in_specs = [blk(3), blk(3), blk(1), blk(1), blk(9)] + [_full(x) for x in ins[5:]]
    out_sd = jax.ShapeDtypeStruct((_N, _H), jnp.float32)
    return pl.pallas_call(
        _enc_nodes_body, grid=grid, in_specs=in_specs,
        out_specs=[blk(_H)] * 3, out_shape=[out_sd] * 3,
        interpret=_INTERPRET,
    )(*ins)


def _enc_edges_body(d, w1p, wdist, wdistw, b1, w2, b2, g, bta, eh_o):
    dd = d[...]
    sq = dd * dd
    lane = lax.broadcasted_iota(jnp.int32, (1, _H), 1)
    m0 = (lane < 3).astype(jnp.float32)
    m1 = ((lane >= 3) & (lane < 6)).astype(jnp.float32)
    dist = jnp.sqrt(jnp.sum(sq * m0, axis=1, keepdims=True))
    distw = jnp.sqrt(jnp.sum(sq * m1, axis=1, keepdims=True))
    h = jnp.dot(dd, w1p[...]) + dist * wdist[...] + distw * wdistw[...] + b1[...]
    h = jax.nn.relu(h)
    h2 = jnp.dot(h, w2[...]) + b2[...]
    eh_o[...] = _ln(h2, g[...], bta[...]).astype(jnp.bfloat16)


def _encode_edges(d, w1p, wdist, wdistw, b1, w2, b2, g, bta):
    ep = d.shape[0]
    grid = (ep // _BE,)
    blk = lambda w: pl.BlockSpec((_BE, w), lambda i: (i, 0))
    ins = [d, w1p, wdist, wdistw, b1, w2, b2, g, bta]
    in_specs = [blk(_H)] + [_full(x) for x in ins[1:]]
    return pl.pallas_call(
        _enc_edges_body, grid=grid, in_specs=in_specs,
        out_specs=blk(_H),
        out_shape=jax.ShapeDtypeStruct((ep, _H), jnp.bfloat16),
        interpret=_INTERPRET,
    )(*ins)


def _edge_mlp_body(gsum, eh, w1c, b1, w2, b2, g, bta, ne_o, eo_o):
    ehv = eh[...].astype(jnp.float32)
    h = jax.nn.relu(gsum[...] + jnp.dot(ehv, w1c[...]) + b1[...])
    h2 = jax.nn.relu(jnp.dot(h, w2[...]) + b2[...])
    ne = _ln(h2, g[...], bta[...])
    ne_o[...] = ne
    if eo_o is not None:
        eo_o[...] = (ne + ehv).astype(jnp.bfloat16)


def _edge_mlp(gsum, eh, w1c, b1, w2, b2, g, bta, want_eh_out):
    ep = gsum.shape[0]
    grid = (ep // _BE,)
    blk = pl.BlockSpec((_BE, _H), lambda i: (i, 0))
    ins = [gsum, eh, w1c, b1, w2, b2, g, bta]
    in_specs = [blk, blk] + [_full(x) for x in ins[2:]]
    out_sd = jax.ShapeDtypeStruct((ep, _H), jnp.float32)
    out_sd_bf = jax.ShapeDtypeStruct((ep, _H), jnp.bfloat16)
    if want_eh_out:
        body = _edge_mlp_body
        out_specs, out_shape = [blk, blk], [out_sd, out_sd_bf]
    else:
        body = functools.partial(_edge_mlp_body, eo_o=None)
        out_specs, out_shape = blk, out_sd
    return pl.pallas_call(
        body, grid=grid, in_specs=in_specs,
        out_specs=out_specs, out_shape=out_shape,
        interpret=_INTERPRET,
    )(*ins)


def _sum_refs(refs):
    acc = refs[0][...]
    for r in refs[1:]:
        acc = acc + r[...]
    return acc


def _node_mlp(xh, ps, w1a, w1b, b1, w2, b2, g, bta, wa, wb):
    np_ = len(ps)

    def body(*refs):
        xh_r = refs[0]
        p_refs = refs[1:1 + np_]
        (w1a_r, w1b_r, b1_r, w2_r, b2_r, g_r, bta_r, wa_r, wb_r,
         nx_o, a_o, b_o) = refs[1 + np_:]
        xhv = xh_r[...]
        aggr = _sum_refs(p_refs)
        h = jax.nn.relu(jnp.dot(xhv, w1a_r[...]) + jnp.dot(aggr, w1b_r[...])
                        + b1_r[...])
        h2 = jax.nn.relu(jnp.dot(h, w2_r[...]) + b2_r[...])
        nx = _ln(h2, g_r[...], bta_r[...]) + xhv
        nx_o[...] = nx
        a_o[...] = jnp.dot(nx, wa_r[...])
        b_o[...] = jnp.dot(nx, wb_r[...])

    grid = (_N // _BN,)
    blk = pl.BlockSpec((_BN, _H), lambda i: (i, 0))
    ins = [xh] + list(ps) + [w1a, w1b, b1, w2, b2, g, bta, wa, wb]
    in_specs = [blk] * (1 + np_) + [_full(x) for x in ins[1 + np_:]]
    out_sd = jax.ShapeDtypeStruct((_N, _H), jnp.float32)
    return pl.pallas_call(
        body, grid=grid, in_specs=in_specs,
        out_specs=[blk] * 3, out_shape=[out_sd] * 3,
        interpret=_INTERPRET,
    )(*ins)


def _node_dec(xh, ps, w1a, w1b, b1, w2, b2, g, bta, wd1, bd1, wd2, bd2):
    np_ = len(ps)

    def body(*refs):
        xh_r = refs[0]
        p_refs = refs[1:1 + np_]
        (w1a_r, w1b_r, b1_r, w2_r, b2_r, g_r, bta_r,
         wd1_r, bd1_r, wd2_r, bd2_r, out_o) = refs[1 + np_:]
        xhv = xh_r[...]
        aggr = _sum_refs(p_refs)
        h = jax.nn.relu(jnp.dot(xhv, w1a_r[...]) + jnp.dot(aggr, w1b_r[...])
                        + b1_r[...])
        h2 = jax.nn.relu(jnp.dot(h, w2_r[...]) + b2_r[...])
        nx = _ln(h2, g_r[...], bta_r[...]) + xhv
        dh = jax.nn.relu(jnp.dot(nx, wd1_r[...]) + bd1_r[...])
        out_o[...] = jnp.dot(dh, wd2_r[...]) + bd2_r[...]

    grid = (_N // _BN,)
    blk = pl.BlockSpec((_BN, _H), lambda i: (i, 0))
    ins = [xh] + list(ps) + [w1a, w1b, b1, w2, b2, g, bta, wd1, bd1, wd2, bd2]
    in_specs = [blk] * (1 + np_) + [_full(x) for x in ins[1 + np_:]]
    return pl.pallas_call(
        body, grid=grid, in_specs=in_specs,
        out_specs=pl.BlockSpec((_BN, 3), lambda i: (i, 0)),
        out_shape=jax.ShapeDtypeStruct((_N, 3), jnp.float32),
        interpret=_INTERPRET,
    )(*ins)


_NC = 2
_NS = 16
_NW = _NC * _NS


@functools.lru_cache(maxsize=None)
def _make_gather_combine(subtract, ch, nchunk):
    mesh = plsc.VectorSubcoreMesh(core_axis_name="c", subcore_axis_name="s")
    epw = ch * nchunk
    e_part = epw * _NW

    def body(ta, tb, s3, r3, out, sbuf, rbuf, ra0, rb0, ra1, rb1,
             sa0, sb0, sa1, sb1):
        wid = lax.axis_index("s") * _NC + lax.axis_index("c")
        base0 = wid * epw
        pltpu.sync_copy(s3.at[wid], sbuf)
        pltpu.sync_copy(r3.at[wid], rbuf)

        ras = (ra0, ra1)
        rbs = (rb0, rb1)
        sas = (sa0, sa1)
        sbs = (sb0, sb1)

        def issue(c, p):
            pltpu.async_copy(ta.at[sbuf.at[c]], ras[p], sas[p])
            pltpu.async_copy(tb.at[rbuf.at[c]], rbs[p], sbs[p])

        def process(c, p):
            pltpu.make_async_copy(ta.at[sbuf.at[c]], ras[p], sas[p]).wait()
            pltpu.make_async_copy(tb.at[rbuf.at[c]], rbs[p], sbs[p]).wait()
            ra, rb = ras[p], rbs[p]

            def row(i, carry2):
                for k in range(4):
                    for j in range(_H // 16):
                        sl = pl.ds(j * 16, 16)
                        if subtract:
                            ra[i * 4 + k, sl] = ra[i * 4 + k, sl] - rb[i * 4 + k, sl]
                        else:
                            ra[i * 4 + k, sl] = ra[i * 4 + k, sl] + rb[i * 4 + k, sl]
                return carry2

            lax.fori_loop(0, ch // 4, row, 0)
            pltpu.sync_copy(ra, out.at[pl.ds(base0 + c * ch, ch)])

            @pl.when(c + 2 < nchunk)
            def _():
                issue(c + 2, p)

        issue(0, 0)
        issue(1, 1)

        def pair(i, carry):
            process(i * 2, 0)
            process(i * 2 + 1, 1)
            return carry

        lax.fori_loop(0, nchunk // 2, pair, 0)
        if nchunk % 2:
            process(nchunk - 1, 0)

    return pl.kernel(
        body,
        out_type=jax.ShapeDtypeStruct((e_part, _H), jnp.float32),
        mesh=mesh,
        scratch_types=[
            pltpu.VMEM((nchunk, ch), jnp.int32),
            pltpu.VMEM((nchunk, ch), jnp.int32),
            pltpu.VMEM((ch, _H), jnp.float32),
            pltpu.VMEM((ch, _H), jnp.float32),
            pltpu.VMEM((ch, _H), jnp.float32),
            pltpu.VMEM((ch, _H), jnp.float32),
            pltpu.SemaphoreType.DMA,
            pltpu.SemaphoreType.DMA,
            pltpu.SemaphoreType.DMA,
            pltpu.SemaphoreType.DMA,
        ],
    )


def _gather_combine(ta, tb, s3, r3, subtract):
    nchunk, ch = s3.shape[1], s3.shape[2]
    return _make_gather_combine(subtract, ch, nchunk)(ta, tb, s3, r3)


_DRT = 10
_DRW = _N // _DRT
_ZR = 40


@functools.lru_cache(maxsize=None)
def _make_scatter_add(ch, nchunk):
    mesh = plsc.VectorSubcoreMesh(core_axis_name="c", subcore_axis_name="s")
    epw = ch * nchunk

    def body(rows_hbm, r3, out, acc, ribuf, row_0, row_1, zbuf, sl0, sl1):
        cid = lax.axis_index("c")
        sid = lax.axis_index("s")
        wid = sid * _NC + cid
        base0 = wid * epw
        pltpu.sync_copy(r3.at[wid], ribuf)
        zv = jnp.zeros((16,), jnp.float32)

        def zrow(i, carry):
            for j in range(_H // 16):
                zbuf[i, pl.ds(j * 16, 16)] = zv
            return carry

        lax.fori_loop(0, _ZR, zrow, 0)
        row0 = sid * _DRW

        @pl.when(sid < _DRT)
        def _zero():
            for k in range(_DRW // _ZR):
                pltpu.sync_copy(zbuf, acc.at[pl.ds(row0 + k * _ZR, _ZR)])

        plsc.subcore_barrier()

        rows = (row_0, row_1)
        sems = (sl0, sl1)

        def issue(c, p):
            pltpu.async_copy(rows_hbm.at[pl.ds(base0 + c * ch, ch)],
                             rows[p], sems[p])

        def process(c, p):
            pltpu.make_async_copy(rows_hbm.at[pl.ds(base0 + c * ch, ch)],
                                  rows[p], sems[p]).wait()
            pltpu.sync_copy(rows[p], acc.at[ribuf.at[c]], add=True)

            @pl.when(c + 2 < nchunk)
            def _():
                issue(c + 2, p)

        issue(0, 0)
        issue(1, 1)

        def pair(i, carry):
            process(i * 2, 0)
            process(i * 2 + 1, 1)
            return carry

        lax.fori_loop(0, nchunk // 2, pair, 0)
        if nchunk % 2:
            process(nchunk - 1, 0)
        plsc.subcore_barrier()

        @pl.when(sid < _DRT)
        def _drain():
            for k in range(_DRW // _ZR):
                sl = pl.ds(row0 + k * _ZR, _ZR)
                pltpu.sync_copy(acc.at[sl], zbuf)
                pltpu.sync_copy(zbuf, out.at[pl.ds(cid * _N + row0 + k * _ZR, _ZR)])

    return pl.kernel(
        body,
        out_type=jax.ShapeDtypeStruct((2 * _N, _H), jnp.float32),
        mesh=mesh,
        scratch_types=[
            pltpu.VMEM_SHARED((_N, _H), jnp.float32),
            pltpu.VMEM((nchunk, ch), jnp.int32),
            pltpu.VMEM((ch, _H), jnp.float32),
            pltpu.VMEM((ch, _H), jnp.float32),
            pltpu.VMEM((_ZR, _H), jnp.float32),
            pltpu.SemaphoreType.DMA,
            pltpu.SemaphoreType.DMA,
        ],
    )


def _scatter_add(rows, r3):
    nchunk, ch = r3.shape[1], r3.shape[2]
    p = _make_scatter_add(ch, nchunk)(rows, r3)
    return p[:_N], p[_N:]


def kernel(world_pos, mesh_pos, phi, swelling_phi, node_type, mat_param,
           edge_index, params):
    f32 = jnp.float32
    ep = _E // _PARTS
    epw = ep // _NW
    chp = next(c for c in (80, 40, 16, 8) if epw % c == 0)
    nchp = epw // chp
    s_parts = [edge_index[0, k * ep:(k + 1) * ep].reshape(_NW, nchp, chp)
               for k in range(_PARTS)]
    r_parts = [edge_index[1, k * ep:(k + 1) * ep].reshape(_NW, nchp, chp)
               for k in range(_PARTS)]

    ne = params["node_enc"]
    wn1 = ne["l1"]["w"]
    wu, wphi, wsphi, wnt, wmat = (wn1[0:3], wn1[3:4], wn1[4:5], wn1[5:14],
                                  wn1[14:19])
    ee = params["edge_enc"]
    we1 = ee["l1"]["w"]
    w1p = jnp.concatenate([we1[0:3], we1[4:7], we1[8:9],
                           jnp.zeros((_H - 7, _H), f32)], axis=0)
    wdist, wdistw = we1[3:4], we1[7:8]

    procs = params["procs"]
    ew = [p["edge"] for p in procs]
    nw = [p["node"] for p in procs]
    e_w1a = [w["l1"]["w"][0:_H] for w in ew]
    e_w1b = [w["l1"]["w"][_H:2 * _H] for w in ew]
    e_w1c = [w["l1"]["w"][2 * _H:] for w in ew]
    n_w1a = [w["l1"]["w"][0:_H] for w in nw]
    n_w1b = [w["l1"]["w"][_H:] for w in nw]

    def row(x):
        return x.reshape(1, -1)

    t_tab = jnp.concatenate([mesh_pos, world_pos, phi,
                             jnp.zeros((_N, _H - 7), f32)], axis=1)
    d_parts = [_gather_combine(t_tab, t_tab, s_parts[k], r_parts[k], True)
               for k in range(_PARTS)]

    x_h, a_t, b_t = _encode_nodes(
        world_pos, mesh_pos, phi, swelling_phi, node_type,
        mat_param.reshape(1, 5), wu, wphi, wsphi, wnt, wmat,
        row(ne["l1"]["b"]), ne["l2"]["w"], row(ne["l2"]["b"]),
        row(ne["g"]), row(ne["bta"]), e_w1a[0], e_w1b[0])

    eh_parts = [_encode_edges(d, w1p, wdist, wdistw, row(ee["l1"]["b"]),
                              ee["l2"]["w"], row(ee["l2"]["b"]),
                              row(ee["g"]), row(ee["bta"]))
                for d in d_parts]

    for t in range(3):
        last = t == 2
        g_parts = [_gather_combine(a_t, b_t, s_parts[k], r_parts[k], False)
                   for k in range(_PARTS)]
        ne_parts = []
        new_eh = []
        for k in range(_PARTS):
            if last:
                nek = _edge_mlp(g_parts[k], eh_parts[k], e_w1c[t],
                                row(ew[t]["l1"]["b"]), ew[t]["l2"]["w"],
                                row(ew[t]["l2"]["b"]), row(ew[t]["g"]),
                                row(ew[t]["bta"]), False)
            else:
                nek, ehk = _edge_mlp(g_parts[k], eh_parts[k], e_w1c[t],
                                     row(ew[t]["l1"]["b"]), ew[t]["l2"]["w"],
                                     row(ew[t]["l2"]["b"]), row(ew[t]["g"]),
                                     row(ew[t]["bta"]), True)
                new_eh.append(ehk)
            ne_parts.append(nek)
        eh_parts = new_eh
        ps = []
        for k in range(_PARTS):
            p0, p1 = _scatter_add(ne_parts[k], r_parts[k])
            ps += [p0, p1]
        w = nw[t]
        if last:
            dec = params["dec"]
            out = _node_dec(x_h, ps, n_w1a[t], n_w1b[t],
                            row(w["l1"]["b"]), w["l2"]["w"], row(w["l2"]["b"]),
                            row(w["g"]), row(w["bta"]),
                            dec["l1"]["w"], row(dec["l1"]["b"]),
                            dec["l2"]["w"], row(dec["l2"]["b"]))
        else:
            x_h, a_t, b_t = _node_mlp(
                x_h, ps, n_w1a[t], n_w1b[t],
                row(w["l1"]["b"]), w["l2"]["w"], row(w["l2"]["b"]),
                row(w["g"]), row(w["bta"]), e_w1a[t + 1], e_w1b[t + 1])

    return out[None, :, :]


# speedup vs baseline: 1.6367x; 1.0733x over previous
import functools

import jax
import jax.numpy as jnp
from jax import lax
from jax.experimental import pallas as pl
from jax.experimental.pallas import tpu as pltpu
from jax.experimental.pallas import tpu_sc as plsc

_N = 10000
_E = 320000
_H = 128
_BN = 1000
_BE = 2000
_PARTS = 2


def _ln(h, g, b):
    m = jnp.mean(h, axis=-1, keepdims=True)
    v = jnp.mean((h - m) ** 2, axis=-1, keepdims=True)
    return (h - m) * jax.lax.rsqrt(v + 1e-5) * g + b


def _full(x):
    return pl.BlockSpec(x.shape, lambda i: tuple(0 for _ in x.shape))


def _enc_nodes_body(wp, mp, phi, sphi, nt, mat, wu, wphi, wsphi, wnt, wmat,
                    b1, w2, b2, g, bta, wa, wb, xh_o, a_o, b_o):
    u = wp[...] - mp[...]
    h = (jnp.dot(u, wu[...]) + phi[...] * wphi[...] + sphi[...] * wsphi[...]
         + jnp.dot(nt[...], wnt[...]) + jnp.dot(mat[...], wmat[...]) + b1[...])
    h = jax.nn.relu(h)
    h2 = jnp.dot(h, w2[...]) + b2[...]
    xh = _ln(h2, g[...], bta[...])
    xh_o[...] = xh
    a_o[...] = jnp.dot(xh, wa[...])
    b_o[...] = jnp.dot(xh, wb[...])


def _encode_nodes(wp, mp, phi, sphi, nt, mat, wu, wphi, wsphi, wnt, wmat,
                  b1, w2, b2, g, bta, wa, wb):
    grid = (_N // _BN,)
    blk = lambda d: pl.BlockSpec((_BN, d), lambda i: (i, 0))
    ins = [wp, mp, phi, sphi, nt, mat, wu, wphi, wsphi, wnt, wmat,
           b1, w2, b2, g, bta, wa, wb]
    in_specs = [blk(3), blk(3), blk(1), blk(1), blk(9)] + [_full(x) for x in ins[5:]]
    out_sd = jax.ShapeDtypeStruct((_N, _H), jnp.float32)
    return pl.pallas_call(
        _enc_nodes_body, grid=grid, in_specs=in_specs,
        out_specs=[blk(_H)] * 3, out_shape=[out_sd] * 3,
    )(*ins)


def _enc_edges_body(d, w1p, wdist, wdistw, b1, w2, b2, g, bta, eh_o):
    dd = d[...]
    sq = dd * dd
    lane = lax.broadcasted_iota(jnp.int32, (1, _H), 1)
    m0 = (lane < 3).astype(jnp.float32)
    m1 = ((lane >= 3) & (lane < 6)).astype(jnp.float32)
    dist = jnp.sqrt(jnp.sum(sq * m0, axis=1, keepdims=True))
    distw = jnp.sqrt(jnp.sum(sq * m1, axis=1, keepdims=True))
    h = jnp.dot(dd, w1p[...]) + dist * wdist[...] + distw * wdistw[...] + b1[...]
    h = jax.nn.relu(h)
    h2 = jnp.dot(h, w2[...]) + b2[...]
    eh_o[...] = _ln(h2, g[...], bta[...]).astype(jnp.bfloat16)


def _encode_edges(d, w1p, wdist, wdistw, b1, w2, b2, g, bta):
    ep = d.shape[0]
    grid = (ep // _BE,)
    blk = lambda w: pl.BlockSpec((_BE, w), lambda i: (i, 0))
    ins = [d, w1p, wdist, wdistw, b1, w2, b2, g, bta]
    in_specs = [blk(_H)] + [_full(x) for x in ins[1:]]
    return pl.pallas_call(
        _enc_edges_body, grid=grid, in_specs=in_specs,
        out_specs=blk(_H),
        out_shape=jax.ShapeDtypeStruct((ep, _H), jnp.bfloat16),
    )(*ins)


def _edge_mlp_body(gsum, eh, w1c, b1, w2, b2, g, bta, ne_o, eo_o):
    ehv = eh[...].astype(jnp.float32)
    h = jax.nn.relu(gsum[...] + jnp.dot(ehv, w1c[...]) + b1[...])
    h2 = jax.nn.relu(jnp.dot(h, w2[...]) + b2[...])
    ne = _ln(h2, g[...], bta[...])
    ne_o[...] = ne
    if eo_o is not None:
        eo_o[...] = (ne + ehv).astype(jnp.bfloat16)


def _edge_mlp(gsum, eh, w1c, b1, w2, b2, g, bta, want_eh_out):
    ep = gsum.shape[0]
    grid = (ep // _BE,)
    blk = pl.BlockSpec((_BE, _H), lambda i: (i, 0))
    ins = [gsum, eh, w1c, b1, w2, b2, g, bta]
    in_specs = [blk, blk] + [_full(x) for x in ins[2:]]
    out_sd = jax.ShapeDtypeStruct((ep, _H), jnp.float32)
    out_sd_bf = jax.ShapeDtypeStruct((ep, _H), jnp.bfloat16)
    if want_eh_out:
        body = _edge_mlp_body
        out_specs, out_shape = [blk, blk], [out_sd, out_sd_bf]
    else:
        body = functools.partial(_edge_mlp_body, eo_o=None)
        out_specs, out_shape = blk, out_sd
    return pl.pallas_call(
        body, grid=grid, in_specs=in_specs,
        out_specs=out_specs, out_shape=out_shape,
    )(*ins)


def _sum_refs(refs):
    acc = refs[0][...]
    for r in refs[1:]:
        acc = acc + r[...]
    return acc


def _node_mlp(xh, ps, w1a, w1b, b1, w2, b2, g, bta, wa, wb):
    np_ = len(ps)

    def body(*refs):
        xh_r = refs[0]
        p_refs = refs[1:1 + np_]
        (w1a_r, w1b_r, b1_r, w2_r, b2_r, g_r, bta_r, wa_r, wb_r,
         nx_o, a_o, b_o) = refs[1 + np_:]
        xhv = xh_r[...]
        aggr = _sum_refs(p_refs)
        h = jax.nn.relu(jnp.dot(xhv, w1a_r[...]) + jnp.dot(aggr, w1b_r[...])
                        + b1_r[...])
        h2 = jax.nn.relu(jnp.dot(h, w2_r[...]) + b2_r[...])
        nx = _ln(h2, g_r[...], bta_r[...]) + xhv
        nx_o[...] = nx
        a_o[...] = jnp.dot(nx, wa_r[...])
        b_o[...] = jnp.dot(nx, wb_r[...])

    grid = (_N // _BN,)
    blk = pl.BlockSpec((_BN, _H), lambda i: (i, 0))
    ins = [xh] + list(ps) + [w1a, w1b, b1, w2, b2, g, bta, wa, wb]
    in_specs = [blk] * (1 + np_) + [_full(x) for x in ins[1 + np_:]]
    out_sd = jax.ShapeDtypeStruct((_N, _H), jnp.float32)
    return pl.pallas_call(
        body, grid=grid, in_specs=in_specs,
        out_specs=[blk] * 3, out_shape=[out_sd] * 3,
    )(*ins)


def _node_dec(xh, ps, w1a, w1b, b1, w2, b2, g, bta, wd1, bd1, wd2, bd2):
    np_ = len(ps)

    def body(*refs):
        xh_r = refs[0]
        p_refs = refs[1:1 + np_]
        (w1a_r, w1b_r, b1_r, w2_r, b2_r, g_r, bta_r,
         wd1_r, bd1_r, wd2_r, bd2_r, out_o) = refs[1 + np_:]
        xhv = xh_r[...]
        aggr = _sum_refs(p_refs)
        h = jax.nn.relu(jnp.dot(xhv, w1a_r[...]) + jnp.dot(aggr, w1b_r[...])
                        + b1_r[...])
        h2 = jax.nn.relu(jnp.dot(h, w2_r[...]) + b2_r[...])
        nx = _ln(h2, g_r[...], bta_r[...]) + xhv
        dh = jax.nn.relu(jnp.dot(nx, wd1_r[...]) + bd1_r[...])
        out_o[...] = jnp.dot(dh, wd2_r[...]) + bd2_r[...]

    grid = (_N // _BN,)
    blk = pl.BlockSpec((_BN, _H), lambda i: (i, 0))
    ins = [xh] + list(ps) + [w1a, w1b, b1, w2, b2, g, bta, wd1, bd1, wd2, bd2]
    in_specs = [blk] * (1 + np_) + [_full(x) for x in ins[1 + np_:]]
    return pl.pallas_call(
        body, grid=grid, in_specs=in_specs,
        out_specs=pl.BlockSpec((_BN, 3), lambda i: (i, 0)),
        out_shape=jax.ShapeDtypeStruct((_N, 3), jnp.float32),
    )(*ins)


_NC = 2
_NS = 16
_NW = _NC * _NS


@functools.lru_cache(maxsize=None)
def _make_gather_combine(subtract, ch, nchunk):
    mesh = plsc.VectorSubcoreMesh(core_axis_name="c", subcore_axis_name="s")
    epw = ch * nchunk
    e_part = epw * _NW

    def body(ta, tb, s3, r3, out, sbuf, rbuf, ra0, rb0, ra1, rb1,
             sa0, sb0, sa1, sb1):
        wid = lax.axis_index("s") * _NC + lax.axis_index("c")
        base0 = wid * epw
        pltpu.sync_copy(s3.at[wid], sbuf)
        pltpu.sync_copy(r3.at[wid], rbuf)

        ras = (ra0, ra1)
        rbs = (rb0, rb1)
        sas = (sa0, sa1)
        sbs = (sb0, sb1)

        def issue(c, p):
            pltpu.async_copy(ta.at[sbuf.at[c]], ras[p], sas[p])
            pltpu.async_copy(tb.at[rbuf.at[c]], rbs[p], sbs[p])

        def process(c, p):
            pltpu.make_async_copy(ta.at[sbuf.at[c]], ras[p], sas[p]).wait()
            pltpu.make_async_copy(tb.at[rbuf.at[c]], rbs[p], sbs[p]).wait()
            ra, rb = ras[p], rbs[p]

            def row(i, carry2):
                for k in range(4):
                    for j in range(_H // 16):
                        sl = pl.ds(j * 16, 16)
                        if subtract:
                            ra[i * 4 + k, sl] = ra[i * 4 + k, sl] - rb[i * 4 + k, sl]
                        else:
                            ra[i * 4 + k, sl] = ra[i * 4 + k, sl] + rb[i * 4 + k, sl]
                return carry2

            lax.fori_loop(0, ch // 4, row, 0)
            pltpu.sync_copy(ra, out.at[pl.ds(base0 + c * ch, ch)])

            @pl.when(c + 2 < nchunk)
            def _():
                issue(c + 2, p)

        issue(0, 0)
        issue(1, 1)

        def pair(i, carry):
            process(i * 2, 0)
            process(i * 2 + 1, 1)
            return carry

        lax.fori_loop(0, nchunk // 2, pair, 0)
        if nchunk % 2:
            process(nchunk - 1, 0)

    return pl.kernel(
        body,
        out_type=jax.ShapeDtypeStruct((e_part, _H), jnp.float32),
        mesh=mesh,
        scratch_types=[
            pltpu.VMEM((nchunk, ch), jnp.int32),
            pltpu.VMEM((nchunk, ch), jnp.int32),
            pltpu.VMEM((ch, _H), jnp.float32),
            pltpu.VMEM((ch, _H), jnp.float32),
            pltpu.VMEM((ch, _H), jnp.float32),
            pltpu.VMEM((ch, _H), jnp.float32),
            pltpu.SemaphoreType.DMA,
            pltpu.SemaphoreType.DMA,
            pltpu.SemaphoreType.DMA,
            pltpu.SemaphoreType.DMA,
        ],
    )


def _gather_combine(ta, tb, s3, r3, subtract):
    nchunk, ch = s3.shape[1], s3.shape[2]
    return _make_gather_combine(subtract, ch, nchunk)(ta, tb, s3, r3)


_DRT = 10
_DRW = _N // _DRT
_ZR = 40


@functools.lru_cache(maxsize=None)
def _make_scatter_add(ch, nchunk):
    mesh = plsc.VectorSubcoreMesh(core_axis_name="c", subcore_axis_name="s")
    epw = ch * nchunk

    def body(rows_hbm, r3, out, acc, ribuf, row_0, row_1, zbuf, sl0, sl1):
        cid = lax.axis_index("c")
        sid = lax.axis_index("s")
        wid = sid * _NC + cid
        base0 = wid * epw
        pltpu.sync_copy(r3.at[wid], ribuf)
        zv = jnp.zeros((16,), jnp.float32)

        def zrow(i, carry):
            for j in range(_H // 16):
                zbuf[i, pl.ds(j * 16, 16)] = zv
            return carry

        lax.fori_loop(0, _ZR, zrow, 0)
        row0 = sid * _DRW

        @pl.when(sid < _DRT)
        def _zero():
            for k in range(_DRW // _ZR):
                pltpu.sync_copy(zbuf, acc.at[pl.ds(row0 + k * _ZR, _ZR)])

        plsc.subcore_barrier()

        rows = (row_0, row_1)
        sems = (sl0, sl1)

        def issue(c, p):
            pltpu.async_copy(rows_hbm.at[pl.ds(base0 + c * ch, ch)],
                             rows[p], sems[p])

        def process(c, p):
            pltpu.make_async_copy(rows_hbm.at[pl.ds(base0 + c * ch, ch)],
                                  rows[p], sems[p]).wait()
            pltpu.sync_copy(rows[p], acc.at[ribuf.at[c]], add=True)

            @pl.when(c + 2 < nchunk)
            def _():
                issue(c + 2, p)

        issue(0, 0)
        issue(1, 1)

        def pair(i, carry):
            process(i * 2, 0)
            process(i * 2 + 1, 1)
            return carry

        lax.fori_loop(0, nchunk // 2, pair, 0)
        if nchunk % 2:
            process(nchunk - 1, 0)
        plsc.subcore_barrier()

        @pl.when(sid < _DRT)
        def _drain():
            for k in range(_DRW // _ZR):
                sl = pl.ds(row0 + k * _ZR, _ZR)
                pltpu.sync_copy(acc.at[sl], zbuf)
                pltpu.sync_copy(zbuf, out.at[pl.ds(cid * _N + row0 + k * _ZR, _ZR)])

    return pl.kernel(
        body,
        out_type=jax.ShapeDtypeStruct((2 * _N, _H), jnp.float32),
        mesh=mesh,
        scratch_types=[
            pltpu.VMEM_SHARED((_N, _H), jnp.float32),
            pltpu.VMEM((nchunk, ch), jnp.int32),
            pltpu.VMEM((ch, _H), jnp.float32),
            pltpu.VMEM((ch, _H), jnp.float32),
            pltpu.VMEM((_ZR, _H), jnp.float32),
            pltpu.SemaphoreType.DMA,
            pltpu.SemaphoreType.DMA,
        ],
    )


def _scatter_add(rows, r3):
    nchunk, ch = r3.shape[1], r3.shape[2]
    p = _make_scatter_add(ch, nchunk)(rows, r3)
    return p[:_N], p[_N:]


def kernel(world_pos, mesh_pos, phi, swelling_phi, node_type, mat_param,
           edge_index, params):
    f32 = jnp.float32
    ep = _E // _PARTS
    epw = ep // _NW
    chp = next(c for c in (80, 40, 16, 8) if epw % c == 0)
    nchp = epw // chp
    s_parts = [edge_index[0, k * ep:(k + 1) * ep].reshape(_NW, nchp, chp)
               for k in range(_PARTS)]
    r_parts = [edge_index[1, k * ep:(k + 1) * ep].reshape(_NW, nchp, chp)
               for k in range(_PARTS)]

    ne = params["node_enc"]
    wn1 = ne["l1"]["w"]
    wu, wphi, wsphi, wnt, wmat = (wn1[0:3], wn1[3:4], wn1[4:5], wn1[5:14],
                                  wn1[14:19])
    ee = params["edge_enc"]
    we1 = ee["l1"]["w"]
    w1p = jnp.concatenate([we1[0:3], we1[4:7], we1[8:9],
                           jnp.zeros((_H - 7, _H), f32)], axis=0)
    wdist, wdistw = we1[3:4], we1[7:8]

    procs = params["procs"]
    ew = [p["edge"] for p in procs]
    nw = [p["node"] for p in procs]
    e_w1a = [w["l1"]["w"][0:_H] for w in ew]
    e_w1b = [w["l1"]["w"][_H:2 * _H] for w in ew]
    e_w1c = [w["l1"]["w"][2 * _H:] for w in ew]
    n_w1a = [w["l1"]["w"][0:_H] for w in nw]
    n_w1b = [w["l1"]["w"][_H:] for w in nw]

    def row(x):
        return x.reshape(1, -1)

    t_tab = jnp.concatenate([mesh_pos, world_pos, phi,
                             jnp.zeros((_N, _H - 7), f32)], axis=1)
    d_parts = [_gather_combine(t_tab, t_tab, s_parts[k], r_parts[k], True)
               for k in range(_PARTS)]

    x_h, a_t, b_t = _encode_nodes(
        world_pos, mesh_pos, phi, swelling_phi, node_type,
        mat_param.reshape(1, 5), wu, wphi, wsphi, wnt, wmat,
        row(ne["l1"]["b"]), ne["l2"]["w"], row(ne["l2"]["b"]),
        row(ne["g"]), row(ne["bta"]), e_w1a[0], e_w1b[0])

    eh_parts = [_encode_edges(d, w1p, wdist, wdistw, row(ee["l1"]["b"]),
                              ee["l2"]["w"], row(ee["l2"]["b"]),
                              row(ee["g"]), row(ee["bta"]))
                for d in d_parts]

    for t in range(3):
        last = t == 2
        g_parts = [_gather_combine(a_t, b_t, s_parts[k], r_parts[k], False)
                   for k in range(_PARTS)]
        ne_parts = []
        new_eh = []
        for k in range(_PARTS):
            if last:
                nek = _edge_mlp(g_parts[k], eh_parts[k], e_w1c[t],
                                row(ew[t]["l1"]["b"]), ew[t]["l2"]["w"],
                                row(ew[t]["l2"]["b"]), row(ew[t]["g"]),
                                row(ew[t]["bta"]), False)
            else:
                nek, ehk = _edge_mlp(g_parts[k], eh_parts[k],
                                     e_w1c[t], row(ew[t]["l1"]["b"]),
                                     ew[t]["l2"]["w"], row(ew[t]["l2"]["b"]),
                                     row(ew[t]["g"]), row(ew[t]["bta"]), True)
                new_eh.append(ehk)
            ne_parts.append(nek)
        eh_parts = new_eh
        ps = []
        for k in range(_PARTS):
            p0, p1 = _scatter_add(ne_parts[k], r_parts[k])
            ps += [p0, p1]
        w = nw[t]
        if last:
            dec = params["dec"]
            out = _node_dec(x_h, ps, n_w1a[t], n_w1b[t],
                            row(w["l1"]["b"]), w["l2"]["w"], row(w["l2"]["b"]),
                            row(w["g"]), row(w["bta"]),
                            dec["l1"]["w"], row(dec["l1"]["b"]),
                            dec["l2"]["w"], row(dec["l2"]["b"]))
        else:
            x_h, a_t, b_t = _node_mlp(
                x_h, ps, n_w1a[t], n_w1b[t],
                row(w["l1"]["b"]), w["l2"]["w"], row(w["l2"]["b"]),
                row(w["g"]), row(w["bta"]), e_w1a[t + 1], e_w1b[t + 1])

    return out[None, :, :]
